# baseline (speedup 1.0000x reference)
import jax
import jax.numpy as jnp
from jax.experimental import pallas as pl


def kernel(x, row_embed):
    B = x.shape[0]
    W, D = row_embed.shape
    WD = W * D
    BB = 64

    table = row_embed.reshape(1, WD)

    def body(t_ref, o_ref):
        o_ref[...] = jnp.broadcast_to(t_ref[...], (BB, WD))

    out = pl.pallas_call(
        body,
        grid=(B // BB,),
        in_specs=[pl.BlockSpec((1, WD), lambda i: (0, 0))],
        out_specs=pl.BlockSpec((BB, WD), lambda i: (i, 0)),
        out_shape=jax.ShapeDtypeStruct((B, WD), jnp.float32),
    )(table)
    return out.reshape(B, W, D)


# speedup vs baseline: 1.9924x; 1.9924x over previous
import jax
import jax.numpy as jnp
from jax.experimental import pallas as pl
from jax.experimental.pallas import tpu as pltpu


def kernel(x, row_embed):
    B = x.shape[0]
    W, D = row_embed.shape
    R = 64
    N = B // R

    table = jnp.broadcast_to(row_embed[None], (R, W, D))

    def body(t_ref, o_ref, sem):
        for i in range(N):
            pltpu.make_async_copy(t_ref, o_ref.at[pl.ds(i * R, R)], sem).start()
        for i in range(N):
            pltpu.make_async_copy(t_ref, o_ref.at[pl.ds(i * R, R)], sem).wait()

    return pl.pallas_call(
        body,
        in_specs=[pl.BlockSpec(memory_space=pltpu.MemorySpace.VMEM)],
        out_specs=pl.BlockSpec(memory_space=pl.ANY),
        out_shape=jax.ShapeDtypeStruct((B, W, D), jnp.float32),
        scratch_shapes=[pltpu.SemaphoreType.DMA],
    )(table)
